# baseline (speedup 1.0000x reference)
import functools

import jax
import jax.numpy as jnp
from jax import lax
from jax.experimental import pallas as pl
from jax.experimental.pallas import tpu as pltpu

_R = 200
_NBUF = 4
_CACHE_CHUNKS = 2


def _fc1_body(x_ref, w1_ref, b1_ref, a_ref):
    a_ref[...] = (
        jnp.dot(x_ref[...], w1_ref[...], preferred_element_type=jnp.float32)
        + b1_ref[...]
    )


def _main_body(a_ref, adj_ref, w2_ref, b2_ref, watt_ref, batt_ref, wcls_ref,
               bcls_ref, out_ref,
               b0, b1, b2, b3, cache_ref, h1_ref, bm_ref,
               s0, s1, s2, s3, semc, *, n, h):
    r = _R
    bufs = (b0, b1, b2, b3)
    sems = (s0, s1, s2, s3)
    nring = (n - _CACHE_CHUNKS * r) // r
    cbase = nring * r
    f32 = jnp.float32

    def start_rs(row0, k):
        pltpu.make_async_copy(
            adj_ref.at[pl.ds(row0, r), :], bufs[k], sems[k]).start()

    def wait_rs(k):
        pltpu.make_async_copy(
            adj_ref.at[pl.ds(0, r), :], bufs[k], sems[k]).wait()

    def start_cache(i):
        pltpu.make_async_copy(
            adj_ref.at[pl.ds(cbase + i * r, r), :],
            cache_ref.at[pl.ds(i * r, r), :], semc).start()

    def wait_cache(i):
        pltpu.make_async_copy(
            adj_ref.at[pl.ds(cbase + i * r, r), :],
            cache_ref.at[pl.ds(i * r, r), :], semc).wait()

    def p1_compute(row0, blk):
        h1 = jnp.maximum(
            jnp.dot(blk, a_ref[...], preferred_element_type=f32), 0.0
        )
        h1_ref[pl.ds(row0, r), :] = h1
        bm_ref[pl.ds(row0, r), :] = (
            jnp.dot(h1, w2_ref[...], preferred_element_type=f32) + b2_ref[...]
        )

    def p2_compute(row0, blk, carry):
        m, d, g = carry
        h2 = jnp.maximum(
            jnp.dot(blk, bm_ref[...], preferred_element_type=f32), 0.0
        )
        hrow = h1_ref[pl.ds(row0, r), :] + h2
        s = (
            jnp.dot(hrow, watt_ref[...], preferred_element_type=f32)
            + batt_ref[0, 0]
        )
        m2 = jnp.maximum(m, jnp.max(s))
        sc = jnp.exp(m - m2)
        e = jnp.exp(s - m2)
        return (m2, d * sc + jnp.sum(e),
                g * sc + jnp.sum(e * hrow, axis=0, keepdims=True))

    for k in range(_NBUF):
        start_rs(k * r, k)

    def body1(i, _):
        row0 = _NBUF * i * r
        for k in range(_NBUF):
            wait_rs(k)
            p1_compute(row0 + k * r, bufs[k][...])
            start_rs(row0 + (k + _NBUF) * r, k)
        return 0

    lax.fori_loop(0, nring // _NBUF - 2, body1, 0)

    base = (nring - 8) * r
    for k in range(_NBUF):
        wait_rs(k)
        p1_compute(base + k * r, bufs[k][...])
        start_rs(base + (k + _NBUF) * r, k)
    base = (nring - 4) * r
    wait_rs(0)
    p1_compute(base, b0[...])
    start_cache(0)
    wait_rs(1)
    p1_compute(base + r, b1[...])
    start_cache(1)
    wait_rs(2)
    p1_compute(base + 2 * r, b2[...])
    start_rs(0, 0)
    wait_rs(3)
    p1_compute(base + 3 * r, b3[...])
    start_rs(r, 1)

    for i in range(_CACHE_CHUNKS):
        wait_cache(i)
        p1_compute(cbase + i * r, cache_ref[pl.ds(i * r, r), :])
    start_rs(2 * r, 2)
    start_rs(3 * r, 3)

    carry = (jnp.float32(-jnp.inf), jnp.float32(0.0),
             jnp.zeros((1, h), f32))
    for i in range(_CACHE_CHUNKS):
        carry = p2_compute(cbase + i * r, cache_ref[pl.ds(i * r, r), :], carry)

    def body2(i, carry):
        row0 = _NBUF * i * r
        for k in range(_NBUF):
            wait_rs(k)
            carry = p2_compute(row0 + k * r, bufs[k][...], carry)
            start_rs(row0 + (k + _NBUF) * r, k)
        return carry

    carry = lax.fori_loop(0, nring // _NBUF - 1, body2, carry)

    base = (nring - 4) * r
    for k in range(_NBUF):
        wait_rs(k)
        carry = p2_compute(base + k * r, bufs[k][...], carry)

    m, d, g = carry
    out_ref[...] = (
        jnp.dot(g / d, wcls_ref[...], preferred_element_type=f32)
        + bcls_ref[...]
    )


def kernel(x, adj, W1, b1, W2, b2, Watt, batt, Wcls, bcls):
    N, DIN = x.shape
    H = W1.shape[1]
    C = Wcls.shape[1]
    f32 = jnp.float32

    a = pl.pallas_call(
        _fc1_body,
        out_shape=jax.ShapeDtypeStruct((N, H), f32),
    )(x, W1, b1.reshape(1, H))

    vmem = pl.BlockSpec(memory_space=pltpu.MemorySpace.VMEM)
    out = pl.pallas_call(
        functools.partial(_main_body, n=N, h=H),
        in_specs=[vmem, pl.BlockSpec(memory_space=pl.ANY)] + [vmem] * 6,
        out_shape=jax.ShapeDtypeStruct((1, C), f32),
        scratch_shapes=[
            pltpu.VMEM((_R, N), f32),
            pltpu.VMEM((_R, N), f32),
            pltpu.VMEM((_R, N), f32),
            pltpu.VMEM((_R, N), f32),
            pltpu.VMEM((_CACHE_CHUNKS * _R, N), f32),
            pltpu.VMEM((N, H), f32),
            pltpu.VMEM((N, H), f32),
            pltpu.SemaphoreType.DMA,
            pltpu.SemaphoreType.DMA,
            pltpu.SemaphoreType.DMA,
            pltpu.SemaphoreType.DMA,
            pltpu.SemaphoreType.DMA,
        ],
        compiler_params=pltpu.CompilerParams(
            vmem_limit_bytes=64 * 1024 * 1024,
        ),
    )(a, adj, W2, b2.reshape(1, H), Watt, batt.reshape(1, 1), Wcls,
      bcls.reshape(1, C))

    return out.reshape(C)


# speedup vs baseline: 1.0954x; 1.0954x over previous
import functools

import jax
import jax.numpy as jnp
from jax import lax
from jax.experimental import pallas as pl
from jax.experimental.pallas import tpu as pltpu

_ROWS = 400


def _body(x_ref, adj_ref, w1_ref, b1_ref, w2_ref, b2_ref, watt_ref, batt_ref,
          wcls_ref, bcls_ref, out_ref,
          a_ref, h1_ref, bm_ref, m_ref, d_ref, g_ref, *, nblk, r):
    t = pl.program_id(0)

    @pl.when(t == 0)
    def _init():
        a_ref[...] = (
            jnp.dot(x_ref[...], w1_ref[...], preferred_element_type=jnp.float32)
            + b1_ref[...]
        )
        m_ref[0, 0] = -jnp.inf
        d_ref[0, 0] = 0.0
        g_ref[...] = jnp.zeros_like(g_ref)

    @pl.when(t < nblk)
    def _pass1():
        h1 = jnp.maximum(
            jnp.dot(adj_ref[...], a_ref[...], preferred_element_type=jnp.float32),
            0.0,
        )
        h1_ref[pl.ds(t * r, r), :] = h1
        bm_ref[pl.ds(t * r, r), :] = (
            jnp.dot(h1, w2_ref[...], preferred_element_type=jnp.float32)
            + b2_ref[...]
        )

    @pl.when(t >= nblk)
    def _pass2():
        i = t - nblk
        h2 = jnp.maximum(
            jnp.dot(adj_ref[...], bm_ref[...], preferred_element_type=jnp.float32),
            0.0,
        )
        h = h1_ref[pl.ds(i * r, r), :] + h2
        s = (
            jnp.dot(h, watt_ref[...], preferred_element_type=jnp.float32)
            + batt_ref[0, 0]
        )

        m_old = m_ref[0, 0]
        m_new = jnp.maximum(m_old, jnp.max(s))
        scale = jnp.exp(m_old - m_new)
        e = jnp.exp(s - m_new)
        d_ref[0, 0] = d_ref[0, 0] * scale + jnp.sum(e)
        g_ref[...] = g_ref[...] * scale + jnp.sum(e * h, axis=0, keepdims=True)
        m_ref[0, 0] = m_new

        @pl.when(t == 2 * nblk - 1)
        def _fini():
            g = g_ref[...] / d_ref[0, 0]
            out_ref[...] = (
                jnp.dot(g, wcls_ref[...], preferred_element_type=jnp.float32)
                + bcls_ref[...]
            )


def kernel(x, adj, W1, b1, W2, b2, Watt, batt, Wcls, bcls):
    N, DIN = x.shape
    H = W1.shape[1]
    C = Wcls.shape[1]
    R = _ROWS
    nblk = N // R
    f32 = jnp.float32

    const = lambda t: (0, 0)
    out = pl.pallas_call(
        functools.partial(_body, nblk=nblk, r=R),
        grid=(2 * nblk,),
        in_specs=[
            pl.BlockSpec((N, DIN), const),
            pl.BlockSpec((R, N), lambda t: (lax.rem(t, nblk), 0)),
            pl.BlockSpec((DIN, H), const),
            pl.BlockSpec((1, H), const),
            pl.BlockSpec((H, H), const),
            pl.BlockSpec((1, H), const),
            pl.BlockSpec((H, 1), const),
            pl.BlockSpec((1, 1), const),
            pl.BlockSpec((H, C), const),
            pl.BlockSpec((1, C), const),
        ],
        out_specs=pl.BlockSpec((1, C), const),
        out_shape=jax.ShapeDtypeStruct((1, C), f32),
        compiler_params=pltpu.CompilerParams(
            vmem_limit_bytes=60 * 1024 * 1024,
        ),
        scratch_shapes=[
            pltpu.VMEM((N, H), f32),
            pltpu.VMEM((N, H), f32),
            pltpu.VMEM((N, H), f32),
            pltpu.SMEM((1, 1), f32),
            pltpu.SMEM((1, 1), f32),
            pltpu.VMEM((1, H), f32),
        ],
    )(x, adj, W1, b1.reshape(1, H), W2, b2.reshape(1, H), Watt,
      batt.reshape(1, 1), Wcls, bcls.reshape(1, C))

    return out.reshape(C)


# speedup vs baseline: 1.1013x; 1.0054x over previous
import functools

import jax
import jax.numpy as jnp
from jax import lax
from jax.experimental import pallas as pl
from jax.experimental.pallas import tpu as pltpu

_ROWS = 400


def _body(x_ref, adj_ref, w1_ref, b1_ref, w2_ref, b2_ref, watt_ref, batt_ref,
          wcls_ref, bcls_ref, out_ref,
          a_ref, h1_ref, bm_ref, m_ref, d_ref, g_ref, *, nblk, r):
    t = pl.program_id(0)

    @pl.when(t == 0)
    def _init():
        a_ref[...] = (
            jnp.dot(x_ref[...], w1_ref[...], preferred_element_type=jnp.float32)
            + b1_ref[...]
        )
        m_ref[0, 0] = -jnp.inf
        d_ref[0, 0] = 0.0
        g_ref[...] = jnp.zeros_like(g_ref)

    @pl.when(t < nblk)
    def _pass1():
        h1 = jnp.maximum(
            jnp.dot(adj_ref[...], a_ref[...], preferred_element_type=jnp.float32),
            0.0,
        )
        h1_ref[pl.ds(t * r, r), :] = h1
        bm_ref[pl.ds(t * r, r), :] = (
            jnp.dot(h1, w2_ref[...], preferred_element_type=jnp.float32)
            + b2_ref[...]
        )

    @pl.when(t >= nblk)
    def _pass2():
        i = lax.rem(t - 1, nblk)
        h2 = jnp.maximum(
            jnp.dot(adj_ref[...], bm_ref[...], preferred_element_type=jnp.float32),
            0.0,
        )
        h = h1_ref[pl.ds(i * r, r), :] + h2
        s = (
            jnp.dot(h, watt_ref[...], preferred_element_type=jnp.float32)
            + batt_ref[0, 0]
        )

        m_old = m_ref[0, 0]
        m_new = jnp.maximum(m_old, jnp.max(s))
        scale = jnp.exp(m_old - m_new)
        e = jnp.exp(s - m_new)
        d_ref[0, 0] = d_ref[0, 0] * scale + jnp.sum(e)
        g_ref[...] = g_ref[...] * scale + jnp.sum(e * h, axis=0, keepdims=True)
        m_ref[0, 0] = m_new

        @pl.when(t == 2 * nblk - 1)
        def _fini():
            g = g_ref[...] / d_ref[0, 0]
            out_ref[...] = (
                jnp.dot(g, wcls_ref[...], preferred_element_type=jnp.float32)
                + bcls_ref[...]
            )


def kernel(x, adj, W1, b1, W2, b2, Watt, batt, Wcls, bcls):
    N, DIN = x.shape
    H = W1.shape[1]
    C = Wcls.shape[1]
    R = _ROWS
    nblk = N // R
    f32 = jnp.float32

    const = lambda t: (0, 0)
    out = pl.pallas_call(
        functools.partial(_body, nblk=nblk, r=R),
        grid=(2 * nblk,),
        in_specs=[
            pl.BlockSpec((N, DIN), const),
            pl.BlockSpec((R, N),
                         lambda t: (jnp.where(t < nblk, t, lax.rem(t - 1, nblk)),
                                    0)),
            pl.BlockSpec((DIN, H), const),
            pl.BlockSpec((1, H), const),
            pl.BlockSpec((H, H), const),
            pl.BlockSpec((1, H), const),
            pl.BlockSpec((H, 1), const),
            pl.BlockSpec((1, 1), const),
            pl.BlockSpec((H, C), const),
            pl.BlockSpec((1, C), const),
        ],
        out_specs=pl.BlockSpec((1, C), const),
        out_shape=jax.ShapeDtypeStruct((1, C), f32),
        compiler_params=pltpu.CompilerParams(
            vmem_limit_bytes=60 * 1024 * 1024,
        ),
        scratch_shapes=[
            pltpu.VMEM((N, H), f32),
            pltpu.VMEM((N, H), f32),
            pltpu.VMEM((N, H), f32),
            pltpu.SMEM((1, 1), f32),
            pltpu.SMEM((1, 1), f32),
            pltpu.VMEM((1, H), f32),
        ],
    )(x, adj, W1, b1.reshape(1, H), W2, b2.reshape(1, H), Watt,
      batt.reshape(1, 1), Wcls, bcls.reshape(1, C))

    return out.reshape(C)
